# baseline (speedup 1.0000x reference)
import functools

import jax
import jax.numpy as jnp
from jax import lax
from jax.experimental import pallas as pl
from jax.experimental.pallas import tpu as pltpu
from jax.experimental.pallas import tpu_sc as plsc

NC = 2
NS = 16
NW = NC * NS
LANES = 16


def kernel(token_ids, embedding):
    B, S = token_ids.shape
    V, D = embedding.shape
    N = B * S
    BW = B // NW
    FT = D // 8
    BT = BW // 128

    idx_t = token_ids.T.reshape(N).astype(jnp.int32)
    mesh = plsc.VectorSubcoreMesh(core_axis_name="c", subcore_axis_name="s")

    @functools.partial(
        pl.kernel,
        mesh=mesh,
        compiler_params=pltpu.CompilerParams(use_tc_tiling_on_sc=False,
                                             needs_layout_passes=False),
        out_type=jax.ShapeDtypeStruct((S, FT, B // 128, 8, 128), jnp.float32),
        scratch_types=[
            pltpu.VMEM((2, BW), jnp.int32),
            pltpu.VMEM((2, BW, D), jnp.float32),
            pltpu.VMEM((2, FT, BT, 8, 128), jnp.float32),
            pltpu.SemaphoreType.DMA,
            pltpu.SemaphoreType.DMA,
            pltpu.SemaphoreType.DMA,
            pltpu.SemaphoreType.DMA,
        ],
    )
    def gather_kernel(idx_hbm, table_hbm, out_hbm, idx_v, rows_v, til_v,
                      g0, g1, w0, w1):
        wid = lax.axis_index("s") * NC + lax.axis_index("c")
        b0 = wid * BW
        gsems = (g0, g1)
        wsems = (w0, w1)

        def load_idx(t, buf):
            pltpu.sync_copy(idx_hbm.at[pl.ds(t * B + b0, BW)], idx_v.at[buf])

        def start_gather(buf):
            pltpu.async_copy(table_hbm.at[idx_v.at[buf]], rows_v.at[buf], gsems[buf])

        def wait_gather(buf):
            pltpu.make_async_copy(table_hbm.at[idx_v.at[buf]], rows_v.at[buf],
                                  gsems[buf]).wait()

        def start_wb(t, buf):
            pltpu.async_copy(til_v.at[buf],
                             out_hbm.at[t, :, pl.ds(wid * BT, BT)], wsems[buf])

        def wait_wb(t, buf):
            pltpu.make_async_copy(til_v.at[buf],
                                  out_hbm.at[t, :, pl.ds(wid * BT, BT)],
                                  wsems[buf]).wait()

        def transpose_chunk(buf):
            rows = rows_v.at[buf]
            lane = lax.iota(jnp.int32, LANES)
            for bt in range(BT):
                for j in range(128 // LANES):
                    r = bt * 128 + j * LANES + lane
                    for ft in range(FT):
                        for fs in range(8):
                            col = jnp.full((LANES,), ft * 8 + fs, jnp.int32)
                            til_v.at[buf, ft, bt, fs, pl.ds(j * LANES, LANES)][...] = (
                                plsc.load_gather(rows, [r, col]))

        load_idx(0, 0)
        start_gather(0)

        @pl.loop(0, S)
        def _(t):
            buf = lax.rem(t, 2)

            @pl.when(buf == 0)
            def _even():
                wait_gather(0)

                @pl.when(t + 1 < S)
                def _():
                    load_idx(t + 1, 1)
                    start_gather(1)

                @pl.when(t >= 2)
                def _():
                    wait_wb(t - 2, 0)
                transpose_chunk(0)
                start_wb(t, 0)

            @pl.when(buf == 1)
            def _odd():
                wait_gather(1)

                @pl.when(t + 1 < S)
                def _():
                    load_idx(t + 1, 0)
                    start_gather(0)

                @pl.when(t >= 2)
                def _():
                    wait_wb(t - 2, 1)
                transpose_chunk(1)
                start_wb(t, 1)

        wait_wb(S - 2, 0)
        wait_wb(S - 1, 1)

    out5 = gather_kernel(idx_t, embedding)
    out = out5.transpose(2, 4, 0, 1, 3).reshape(B, S, D)
    return out


# speedup vs baseline: 1.3466x; 1.3466x over previous
import functools

import jax
import jax.numpy as jnp
from jax import lax
from jax.experimental import pallas as pl
from jax.experimental.pallas import tpu as pltpu
from jax.experimental.pallas import tpu_sc as plsc

NC = 2
NS = 16
NW = NC * NS
LANES = 16


def kernel(token_ids, embedding):
    B, S = token_ids.shape
    V, D = embedding.shape
    N = B * S
    BW = B // NW
    FT = D // 8
    BT = BW // 128

    idx_t = token_ids.T.reshape(N).astype(jnp.int32)
    mesh = plsc.VectorSubcoreMesh(core_axis_name="c", subcore_axis_name="s")

    @functools.partial(
        pl.kernel,
        mesh=mesh,
        compiler_params=pltpu.CompilerParams(use_tc_tiling_on_sc=False,
                                             needs_layout_passes=False),
        out_type=jax.ShapeDtypeStruct((S, FT, B // 128, 8, 128), jnp.float32),
        scratch_types=[
            pltpu.VMEM((2, BW), jnp.int32),
            pltpu.VMEM((2, BW, D), jnp.float32),
            pltpu.VMEM((2, FT, BT, 8, 128), jnp.float32),
            pltpu.SemaphoreType.DMA,
            pltpu.SemaphoreType.DMA,
            pltpu.SemaphoreType.DMA,
            pltpu.SemaphoreType.DMA,
        ],
    )
    def gather_kernel(idx_hbm, table_hbm, out_hbm, idx_v, rows_v, til_v,
                      g0, g1, w0, w1):
        wid = lax.axis_index("s") * NC + lax.axis_index("c")
        b0 = wid * BW
        gsems = (g0, g1)
        wsems = (w0, w1)

        def load_idx(t, buf):
            pltpu.sync_copy(idx_hbm.at[pl.ds(t * B + b0, BW)], idx_v.at[buf])

        def start_gather(buf):
            pltpu.async_copy(table_hbm.at[idx_v.at[buf]],
                             rows_v.at[buf], gsems[buf])

        def wait_gather(buf):
            pltpu.make_async_copy(table_hbm.at[idx_v.at[buf]],
                                  rows_v.at[buf], gsems[buf]).wait()

        def start_wb(t, buf):
            pltpu.async_copy(til_v.at[buf],
                             out_hbm.at[t, :, pl.ds(wid * BT, BT)], wsems[buf])

        def wait_wb(t, buf):
            pltpu.make_async_copy(til_v.at[buf],
                                  out_hbm.at[t, :, pl.ds(wid * BT, BT)],
                                  wsems[buf]).wait()

        def transpose_chunk(buf):
            rows = rows_v.at[buf]
            lane = lax.iota(jnp.int32, LANES)
            shift = 4
            pend = []

            def drain_one():
                val, (a, bb, c, d) = pend.pop(0)
                til_v.at[buf, a, bb, c, pl.ds(d * LANES, LANES)][...] = val

            for bt in range(BT):
                for j in range(128 // LANES):
                    r = bt * 128 + j * LANES + lane
                    for ft in range(FT):
                        for fs in range(8):
                            col = jnp.full((LANES,), ft * 8 + fs, jnp.int32)
                            pend.append((plsc.load_gather(rows, [r, col]),
                                         (ft, bt, fs, j)))
                            if len(pend) > shift:
                                drain_one()
            while pend:
                drain_one()

        load_idx(0, 0)
        start_gather(0)

        @pl.loop(0, S)
        def _(t):
            buf = lax.rem(t, 2)

            @pl.when(buf == 0)
            def _even():
                wait_gather(0)

                @pl.when(t + 1 < S)
                def _():
                    load_idx(t + 1, 1)
                    start_gather(1)

                @pl.when(t >= 2)
                def _():
                    wait_wb(t - 2, 0)
                transpose_chunk(0)
                start_wb(t, 0)

            @pl.when(buf == 1)
            def _odd():
                wait_gather(1)

                @pl.when(t + 1 < S)
                def _():
                    load_idx(t + 1, 0)
                    start_gather(0)

                @pl.when(t >= 2)
                def _():
                    wait_wb(t - 2, 1)
                transpose_chunk(1)
                start_wb(t, 1)

        wait_wb(S - 2, 0)
        wait_wb(S - 1, 1)

    out5 = gather_kernel(idx_t, embedding)
    out = out5.transpose(2, 4, 0, 1, 3).reshape(B, S, D)
    return out


# speedup vs baseline: 1.3478x; 1.0009x over previous
import functools

import jax
import jax.numpy as jnp
from jax import lax
from jax.experimental import pallas as pl
from jax.experimental.pallas import tpu as pltpu
from jax.experimental.pallas import tpu_sc as plsc

NC = 2
NS = 16
NW = NC * NS
LANES = 16


def kernel(token_ids, embedding):
    B, S = token_ids.shape
    V, D = embedding.shape
    N = B * S
    BW = B // NW
    FT = D // 8
    BT = BW // 128

    idx_t = token_ids.T.reshape(N).astype(jnp.int32)
    mesh = plsc.VectorSubcoreMesh(core_axis_name="c", subcore_axis_name="s")

    @functools.partial(
        pl.kernel,
        mesh=mesh,
        compiler_params=pltpu.CompilerParams(use_tc_tiling_on_sc=False,
                                             needs_layout_passes=False),
        out_type=jax.ShapeDtypeStruct((S, FT, B // 128, 8, 128), jnp.float32),
        scratch_types=[
            pltpu.VMEM((2, BW), jnp.int32),
            pltpu.VMEM((2, BW, D), jnp.float32),
            pltpu.VMEM((2, FT, BT, 8, 128), jnp.float32),
            pltpu.SemaphoreType.DMA,
            pltpu.SemaphoreType.DMA,
            pltpu.SemaphoreType.DMA,
            pltpu.SemaphoreType.DMA,
        ],
    )
    def gather_kernel(idx_hbm, table_hbm, out_hbm, idx_v, rows_v, til_v,
                      g0, g1, w0, w1):
        wid = lax.axis_index("s") * NC + lax.axis_index("c")
        b0 = wid * BW
        gsems = (g0, g1)
        wsems = (w0, w1)

        def load_idx(t, buf):
            pltpu.sync_copy(idx_hbm.at[pl.ds(t * B + b0, BW)], idx_v.at[buf])

        def start_gather(buf):
            pltpu.async_copy(table_hbm.at[idx_v.at[buf]],
                             rows_v.at[buf], gsems[buf])

        def wait_gather(buf):
            pltpu.make_async_copy(table_hbm.at[idx_v.at[buf]],
                                  rows_v.at[buf], gsems[buf]).wait()

        def start_wb(t, buf):
            pltpu.async_copy(til_v.at[buf],
                             out_hbm.at[t, :, pl.ds(wid * BT, BT)], wsems[buf])

        def wait_wb(t, buf):
            pltpu.make_async_copy(til_v.at[buf],
                                  out_hbm.at[t, :, pl.ds(wid * BT, BT)],
                                  wsems[buf]).wait()

        def transpose_chunk(buf):
            rows = rows_v.at[buf]
            lane = lax.iota(jnp.int32, LANES)
            shift = 8
            pend = []

            def drain_one():
                val, (a, bb, c, d) = pend.pop(0)
                til_v.at[buf, a, bb, c, pl.ds(d * LANES, LANES)][...] = val

            for bt in range(BT):
                for j in range(128 // LANES):
                    r = bt * 128 + j * LANES + lane
                    for ft in range(FT):
                        for fs in range(8):
                            col = jnp.full((LANES,), ft * 8 + fs, jnp.int32)
                            pend.append((plsc.load_gather(rows, [r, col]),
                                         (ft, bt, fs, j)))
                            if len(pend) > shift:
                                drain_one()
            while pend:
                drain_one()

        load_idx(0, 0)
        start_gather(0)

        @pl.loop(0, S)
        def _(t):
            buf = lax.rem(t, 2)

            @pl.when(buf == 0)
            def _even():
                wait_gather(0)

                @pl.when(t + 1 < S)
                def _():
                    load_idx(t + 1, 1)
                    start_gather(1)

                @pl.when(t >= 2)
                def _():
                    wait_wb(t - 2, 0)
                transpose_chunk(0)
                start_wb(t, 0)

            @pl.when(buf == 1)
            def _odd():
                wait_gather(1)

                @pl.when(t + 1 < S)
                def _():
                    load_idx(t + 1, 0)
                    start_gather(0)

                @pl.when(t >= 2)
                def _():
                    wait_wb(t - 2, 1)
                transpose_chunk(1)
                start_wb(t, 1)

        wait_wb(S - 2, 0)
        wait_wb(S - 1, 1)

    out5 = gather_kernel(idx_t, embedding)
    out = out5.transpose(2, 4, 0, 1, 3).reshape(B, S, D)
    return out
